# baseline (speedup 1.0000x reference)
import functools

import jax
import jax.numpy as jnp
from jax import lax
from jax.experimental import pallas as pl
from jax.experimental.pallas import tpu as pltpu
from jax.experimental.pallas import tpu_sc as plsc

VOCAB = 1000
ALIGNED = 896
TAIL = VOCAB - ALIGNED
BATCH = 4096
SEQ = 20


@jax.jit
def _lookup(x, embeddings):
    info = plsc.get_sparse_core_info()
    nw = info.num_cores * info.num_subcores
    b_per_w = BATCH // nw
    n_groups = b_per_w // 2

    table_main = embeddings[:, :ALIGNED]
    table_tail = jnp.pad(embeddings[:, ALIGNED:], ((0, 0), (0, 128 - TAIL)))

    mesh = plsc.VectorSubcoreMesh(core_axis_name="c", subcore_axis_name="s")

    @functools.partial(
        pl.kernel,
        mesh=mesh,
        out_type=jax.ShapeDtypeStruct((BATCH, SEQ, VOCAB), jnp.float32),
        scratch_types=[
            pltpu.VMEM((b_per_w, 24), jnp.int32),
            pltpu.VMEM((SEQ, VOCAB), jnp.float32),
            pltpu.VMEM((SEQ, VOCAB), jnp.float32),
            pltpu.VMEM((8, ALIGNED), jnp.float32),
            pltpu.VMEM((8, ALIGNED), jnp.float32),
            pltpu.VMEM((24, 128), jnp.float32),
            pltpu.VMEM((24, 128), jnp.float32),
            pltpu.SemaphoreType.DMA,
            pltpu.SemaphoreType.DMA,
        ],
    )
    def k(tmain_hbm, ttail_hbm, idx_hbm, out_hbm,
          idx_v, stage0, stage1, rest0, rest1, tail0, tail1, sem0, sem1):
        wid = lax.axis_index("s") * info.num_cores + lax.axis_index("c")
        base = wid * b_per_w
        pltpu.sync_copy(idx_hbm.at[pl.ds(base, b_per_w)], idx_v)

        stages = (stage0, stage1)
        rests = (rest0, rest1)
        tails = (tail0, tail1)
        sems = (sem0, sem1)

        def fire(c, b):
            idx = idx_v.at[c]
            pltpu.async_copy(
                tmain_hbm.at[idx.at[pl.ds(0, 16)]],
                stages[b].at[pl.ds(0, 16), pl.ds(0, ALIGNED)],
                sems[b],
            )
            pltpu.async_copy(
                tmain_hbm.at[idx.at[pl.ds(16, 8)]], rests[b], sems[b]
            )
            pltpu.async_copy(ttail_hbm.at[idx], tails[b], sems[b])

        def drain(c, b):
            idx = idx_v.at[c]
            pltpu.make_async_copy(
                tmain_hbm.at[idx.at[pl.ds(0, 16)]],
                stages[b].at[pl.ds(0, 16), pl.ds(0, ALIGNED)],
                sems[b],
            ).wait()
            pltpu.make_async_copy(
                tmain_hbm.at[idx.at[pl.ds(16, 8)]], rests[b], sems[b]
            ).wait()
            pltpu.make_async_copy(ttail_hbm.at[idx], tails[b], sems[b]).wait()

        for b in range(2):
            fire(b, b)

        def body(g, carry):
            for b in range(2):
                c = g * 2 + b
                drain(c, b)
                for r in range(4):
                    for j in range(ALIGNED // 16):
                        stages[b][16 + r, pl.ds(16 * j, 16)] = (
                            rests[b][r, pl.ds(16 * j, 16)]
                        )
                for r in range(SEQ):
                    stages[b][r, pl.ds(VOCAB - 16, 16)] = (
                        tails[b][r, pl.ds(TAIL - 16, 16)]
                    )
                    for j in range(6):
                        stages[b][r, pl.ds(ALIGNED + 16 * j, 16)] = (
                            tails[b][r, pl.ds(16 * j, 16)]
                        )
                pltpu.sync_copy(stages[b], out_hbm.at[base + c])

                @pl.when(g < n_groups - 1)
                def _():
                    fire(c + 2, b)
            return carry

        lax.fori_loop(0, n_groups, body, 0)

    x24 = jnp.pad(x, ((0, 0), (0, 24 - SEQ)))
    return k(table_main, table_tail, x24)


def kernel(x, embeddings):
    return _lookup(x.astype(jnp.int32), embeddings)


# speedup vs baseline: 1.2054x; 1.2054x over previous
import functools

import jax
import jax.numpy as jnp
from jax import lax
from jax.experimental import pallas as pl
from jax.experimental.pallas import tpu as pltpu
from jax.experimental.pallas import tpu_sc as plsc

VOCAB = 1000
ALIGNED = 896
TAIL = VOCAB - ALIGNED
BATCH = 4096
SEQ = 20


def _lookup(x, embeddings):
    info = plsc.get_sparse_core_info()
    nw = info.num_cores * info.num_subcores
    b_per_w = BATCH // nw
    n_groups = b_per_w // 2

    table_main = embeddings[:, :ALIGNED]
    table_tail = jnp.pad(embeddings[:, ALIGNED:], ((0, 0), (0, 128 - TAIL)))

    mesh = plsc.VectorSubcoreMesh(core_axis_name="c", subcore_axis_name="s")

    @functools.partial(
        pl.kernel,
        mesh=mesh,
        out_type=jax.ShapeDtypeStruct((BATCH, SEQ, VOCAB), jnp.float32),
        scratch_types=[
            pltpu.VMEM((b_per_w, 24), jnp.int32),
            pltpu.VMEM((SEQ, VOCAB), jnp.float32),
            pltpu.VMEM((SEQ, VOCAB), jnp.float32),
            pltpu.VMEM((8, ALIGNED), jnp.float32),
            pltpu.VMEM((8, ALIGNED), jnp.float32),
            pltpu.VMEM((24, 128), jnp.float32),
            pltpu.VMEM((24, 128), jnp.float32),
            pltpu.SemaphoreType.DMA,
            pltpu.SemaphoreType.DMA,
        ],
    )
    def k(tmain_hbm, ttail_hbm, idx_hbm, out_hbm,
          idx_v, stage0, stage1, rest0, rest1, tail0, tail1, sem0, sem1):
        wid = lax.axis_index("s") * info.num_cores + lax.axis_index("c")
        base = wid * b_per_w
        pltpu.sync_copy(idx_hbm.at[pl.ds(base, b_per_w)], idx_v)

        stages = (stage0, stage1)
        rests = (rest0, rest1)
        tails = (tail0, tail1)
        sems = (sem0, sem1)

        def fire(c, b):
            idx = idx_v.at[c]
            pltpu.async_copy(
                tmain_hbm.at[idx.at[pl.ds(0, 16)]],
                stages[b].at[pl.ds(0, 16), pl.ds(0, ALIGNED)],
                sems[b],
            )
            pltpu.async_copy(
                tmain_hbm.at[idx.at[pl.ds(16, 8)]], rests[b], sems[b]
            )
            pltpu.async_copy(ttail_hbm.at[idx], tails[b], sems[b])

        def drain(c, b):
            idx = idx_v.at[c]
            pltpu.make_async_copy(
                tmain_hbm.at[idx.at[pl.ds(0, 16)]],
                stages[b].at[pl.ds(0, 16), pl.ds(0, ALIGNED)],
                sems[b],
            ).wait()
            pltpu.make_async_copy(
                tmain_hbm.at[idx.at[pl.ds(16, 8)]], rests[b], sems[b]
            ).wait()
            pltpu.make_async_copy(ttail_hbm.at[idx], tails[b], sems[b]).wait()

        for b in range(2):
            fire(b, b)

        def body(g, carry):
            for b in range(2):
                c = g * 2 + b
                drain(c, b)
                for r in range(4):
                    for j in range(ALIGNED // 16):
                        stages[b][16 + r, pl.ds(16 * j, 16)] = (
                            rests[b][r, pl.ds(16 * j, 16)]
                        )
                for r in range(SEQ):
                    stages[b][r, pl.ds(VOCAB - 16, 16)] = (
                        tails[b][r, pl.ds(TAIL - 16, 16)]
                    )
                    for j in range(6):
                        stages[b][r, pl.ds(ALIGNED + 16 * j, 16)] = (
                            tails[b][r, pl.ds(16 * j, 16)]
                        )
                plsc.subcore_barrier()
                pltpu.sync_copy(stages[b], out_hbm.at[base + c])

                @pl.when(g < n_groups - 1)
                def _():
                    fire(c + 2, b)
            return carry

        lax.fori_loop(0, n_groups, body, 0)

    x24 = jnp.pad(x, ((0, 0), (0, 24 - SEQ)))
    return k(table_main, table_tail, x24)


def kernel(x, embeddings):
    return _lookup(x.astype(jnp.int32), embeddings)


# speedup vs baseline: 1.2760x; 1.0586x over previous
import functools

import jax
import jax.numpy as jnp
from jax import lax
from jax.experimental import pallas as pl
from jax.experimental.pallas import tpu as pltpu
from jax.experimental.pallas import tpu_sc as plsc

VOCAB = 1000
BATCH = 4096
SEQ = 20


def _lookup(x, embeddings):
    info = plsc.get_sparse_core_info()
    nw = info.num_cores * info.num_subcores
    b_per_w = BATCH // nw
    n_groups = b_per_w // 2

    table8 = jnp.pad(embeddings, ((0, 0), (0, 1024 - VOCAB)))
    table8 = table8.reshape(VOCAB, 8, 128)

    mesh = plsc.VectorSubcoreMesh(core_axis_name="c", subcore_axis_name="s")

    @functools.partial(
        pl.kernel,
        mesh=mesh,
        out_type=jax.ShapeDtypeStruct((BATCH, SEQ, VOCAB), jnp.float32),
        scratch_types=[
            pltpu.VMEM((b_per_w, 24), jnp.int32),
            pltpu.VMEM((24, 8, 128), jnp.float32),
            pltpu.VMEM((24, 8, 128), jnp.float32),
            pltpu.VMEM((SEQ, VOCAB), jnp.float32),
            pltpu.SemaphoreType.DMA,
            pltpu.SemaphoreType.DMA,
        ],
    )
    def k(table_hbm, idx_hbm, out_hbm, idx_v, buf0, buf1, stage, sem0, sem1):
        wid = lax.axis_index("s") * info.num_cores + lax.axis_index("c")
        base = wid * b_per_w
        pltpu.sync_copy(idx_hbm.at[pl.ds(base, b_per_w)], idx_v)

        bufs = (buf0, buf1)
        sems = (sem0, sem1)

        for b in range(2):
            pltpu.async_copy(table_hbm.at[idx_v.at[b]], bufs[b], sems[b])

        def body(g, carry):
            for b in range(2):
                c = g * 2 + b
                pltpu.make_async_copy(
                    table_hbm.at[idx_v.at[c]], bufs[b], sems[b]
                ).wait()
                for r in range(SEQ):
                    stage[r, pl.ds(VOCAB - 16, 16)] = (
                        bufs[b][r, 7, pl.ds(VOCAB - 16 - 896, 16)]
                    )
                    for j in range(62):
                        stage[r, pl.ds(16 * j, 16)] = (
                            bufs[b][r, j // 8, pl.ds(16 * (j % 8), 16)]
                        )
                plsc.subcore_barrier()
                pltpu.sync_copy(stage, out_hbm.at[base + c])

                @pl.when(g < n_groups - 1)
                def _():
                    pltpu.async_copy(
                        table_hbm.at[idx_v.at[c + 2]], bufs[b], sems[b]
                    )
            return carry

        lax.fori_loop(0, n_groups, body, 0)

    x24 = jnp.pad(x, ((0, 0), (0, 24 - SEQ)))
    return k(table8, x24)


def kernel(x, embeddings):
    return _lookup(x.astype(jnp.int32), embeddings)


# speedup vs baseline: 2.0150x; 1.5792x over previous
import functools

import jax
import jax.numpy as jnp
from jax import lax
from jax.experimental import pallas as pl
from jax.experimental.pallas import tpu as pltpu
from jax.experimental.pallas import tpu_sc as plsc

VOCAB = 1000
BATCH = 4096
SEQ = 20


@jax.jit
def _lookup(x, embeddings):
    info = plsc.get_sparse_core_info()
    nw = info.num_cores * info.num_subcores
    b_per_w = BATCH // nw
    n_groups = b_per_w // 2

    mesh = plsc.VectorSubcoreMesh(core_axis_name="c", subcore_axis_name="s")

    @functools.partial(
        pl.kernel,
        mesh=mesh,
        out_type=jax.ShapeDtypeStruct((BATCH, SEQ, VOCAB), jnp.float32),
        scratch_types=[
            pltpu.VMEM((b_per_w, SEQ), jnp.int32),
            pltpu.VMEM((SEQ, VOCAB), jnp.float32),
            pltpu.VMEM((SEQ, VOCAB), jnp.float32),
            pltpu.SemaphoreType.DMA,
            pltpu.SemaphoreType.DMA,
        ],
        compiler_params=pltpu.CompilerParams(use_tc_tiling_on_sc=False),
    )
    def k(table_hbm, idx_hbm, out_hbm, idx_v, rows0, rows1, sem0, sem1):
        wid = lax.axis_index("s") * info.num_cores + lax.axis_index("c")
        base = wid * b_per_w
        pltpu.sync_copy(idx_hbm.at[pl.ds(base, b_per_w)], idx_v)

        bufs = (rows0, rows1)
        sems = (sem0, sem1)

        for b in range(2):
            pltpu.async_copy(table_hbm.at[idx_v.at[b]], bufs[b], sems[b])

        def body(g, carry):
            for b in range(2):
                c = g * 2 + b
                pltpu.make_async_copy(
                    table_hbm.at[idx_v.at[c]], bufs[b], sems[b]
                ).wait()
                pltpu.sync_copy(bufs[b], out_hbm.at[base + c])

                @pl.when(g < n_groups - 1)
                def _():
                    pltpu.async_copy(
                        table_hbm.at[idx_v.at[c + 2]], bufs[b], sems[b]
                    )
            return carry

        lax.fori_loop(0, n_groups, body, 0)

    return k(embeddings, x)


def kernel(x, embeddings):
    return _lookup(x.astype(jnp.int32), embeddings)
